# baseline (speedup 1.0000x reference)
import jax
import jax.numpy as jnp
from jax import lax
from jax.experimental import pallas as pl
from jax.experimental.pallas import tpu as pltpu

RELATION_THRESHOLD = 0.5
_BN = 1000


def _main_body(orig_ref, size_ref, tri_ref, scores_ref, sbox_ref, obox_ref,
               scat_ref, ocat_ref, verb_ref, boxes_ref, labels_ref, keep_ref,
               subids_ref, nkeep_ref, cnt_ref):
    b = pl.program_id(0)
    i = pl.program_id(1)

    @pl.when(i == 0)
    def _init():
        cnt_ref[0] = 0

    x = scores_ref[0]
    m = jnp.max(x, axis=-1, keepdims=True)
    s = jnp.sum(jnp.exp(x - m), axis=-1, keepdims=True)
    keep2 = (1.0 / s) >= RELATION_THRESHOLD
    verb_ref[0] = jnp.where(keep2, x, 0.0)

    kf = keep2.astype(jnp.float32)
    csum = jnp.dot(tri_ref[...], kf, preferred_element_type=jnp.float32)
    base = cnt_ref[0]
    rank = base + csum.astype(jnp.int32) - 1
    ids = jnp.where(keep2, rank, -1)
    subids_ref[0, 0, 0] = ids.reshape(1, _BN)[0]
    keep_ref[0, 0, 0] = keep2.astype(jnp.int32).reshape(1, _BN)[0]
    total = base + jnp.sum(kf).astype(jnp.int32)
    cnt_ref[0] = total
    nkeep_ref[0, 0, 0] = total

    sw = orig_ref[b, 1] / size_ref[b, 1]
    sh = orig_ref[b, 0] / size_ref[b, 0]
    lidx = lax.broadcasted_iota(jnp.int32, (_BN, 4), 1)
    scale = jnp.where((lidx % 2) == 0, sw, sh)
    boxes_ref[0, 0] = sbox_ref[0] * scale
    boxes_ref[0, 1] = obox_ref[0] * scale
    labels_ref[0, 0, 0, 0] = scat_ref[0, 0, 0] - 1
    labels_ref[0, 1, 0, 0] = ocat_ref[0, 0, 0] - 1


def _objids_body(keep_ref, subids_ref, nkeep_ref, objids_ref):
    b = pl.program_id(0)
    nk = nkeep_ref[b, 0, 0]
    k = keep_ref[0, 0, 0]
    objids_ref[0, 0, 0] = jnp.where(k > 0, subids_ref[0, 0, 0] + nk, -1)


def kernel(relation_scores, subject_bbox, object_bbox, subject_category,
           object_category, orig_size, size):
    B, N, C = relation_scores.shape
    nb = N // _BN
    scat4 = subject_category.reshape(B, nb, 1, _BN)
    ocat4 = object_category.reshape(B, nb, 1, _BN)
    row = lax.broadcasted_iota(jnp.int32, (_BN, _BN), 0)
    col = lax.broadcasted_iota(jnp.int32, (_BN, _BN), 1)
    tri = (col <= row).astype(jnp.float32)

    grid = (B, nb)
    verb, boxes4, labels3, keep3, subids3, nkeep = pl.pallas_call(
        _main_body,
        grid=grid,
        in_specs=[
            pl.BlockSpec(memory_space=pltpu.SMEM),
            pl.BlockSpec(memory_space=pltpu.SMEM),
            pl.BlockSpec((_BN, _BN), lambda b, i: (0, 0)),
            pl.BlockSpec((1, _BN, C), lambda b, i: (b, i, 0)),
            pl.BlockSpec((1, _BN, 4), lambda b, i: (b, i, 0)),
            pl.BlockSpec((1, _BN, 4), lambda b, i: (b, i, 0)),
            pl.BlockSpec((1, 1, 1, _BN), lambda b, i: (b, i, 0, 0)),
            pl.BlockSpec((1, 1, 1, _BN), lambda b, i: (b, i, 0, 0)),
        ],
        out_specs=[
            pl.BlockSpec((1, _BN, C), lambda b, i: (b, i, 0)),
            pl.BlockSpec((1, 2, _BN, 4), lambda b, i: (b, 0, i, 0)),
            pl.BlockSpec((1, 2, 1, 1, _BN), lambda b, i: (b, 0, i, 0, 0)),
            pl.BlockSpec((1, 1, 1, _BN), lambda b, i: (b, i, 0, 0)),
            pl.BlockSpec((1, 1, 1, _BN), lambda b, i: (b, i, 0, 0)),
            pl.BlockSpec((1, 1, 1), lambda b, i: (b, 0, 0),
                         memory_space=pltpu.SMEM),
        ],
        out_shape=[
            jax.ShapeDtypeStruct((B, N, C), jnp.float32),
            jax.ShapeDtypeStruct((B, 2, N, 4), jnp.float32),
            jax.ShapeDtypeStruct((B, 2, nb, 1, _BN), jnp.int32),
            jax.ShapeDtypeStruct((B, nb, 1, _BN), jnp.int32),
            jax.ShapeDtypeStruct((B, nb, 1, _BN), jnp.int32),
            jax.ShapeDtypeStruct((B, 1, 1), jnp.int32),
        ],
        scratch_shapes=[pltpu.SMEM((1,), jnp.int32)],
        compiler_params=pltpu.CompilerParams(
            dimension_semantics=("arbitrary", "arbitrary")),
    )(orig_size, size, tri, relation_scores, subject_bbox, object_bbox,
      scat4, ocat4)

    objids3 = pl.pallas_call(
        _objids_body,
        grid=grid,
        in_specs=[
            pl.BlockSpec((1, 1, 1, _BN), lambda b, i: (b, i, 0, 0)),
            pl.BlockSpec((1, 1, 1, _BN), lambda b, i: (b, i, 0, 0)),
            pl.BlockSpec(memory_space=pltpu.SMEM),
        ],
        out_specs=pl.BlockSpec((1, 1, 1, _BN), lambda b, i: (b, i, 0, 0)),
        out_shape=jax.ShapeDtypeStruct((B, nb, 1, _BN), jnp.int32),
        compiler_params=pltpu.CompilerParams(
            dimension_semantics=("arbitrary", "arbitrary")),
    )(keep3, subids3, nkeep)

    boxes = boxes4.reshape(B, 2 * N, 4)
    labels = labels3.reshape(B, 2 * N)
    keep = keep3.reshape(B, N).astype(bool)
    sub_ids = subids3.reshape(B, N)
    obj_ids = objids3.reshape(B, N)
    return boxes, labels, verb, keep, sub_ids, obj_ids
